# Initial kernel scaffold; baseline (speedup 1.0000x reference)
import jax
import jax.numpy as jnp
from jax.experimental import pallas as pl


def kernel(logits, label):
    raise NotImplementedError("write your pallas kernel here")


# speedup vs baseline: 1.2663x; 1.2663x over previous
import functools

import jax
import jax.numpy as jnp
from jax.experimental import pallas as pl
from jax.experimental.pallas import tpu as pltpu


def _prod_rows(t):
    C = t.shape[0]
    if C == 1:
        return t
    h = C // 2
    r = _prod_rows(t[0:h] * t[h:2 * h])
    if C % 2:
        r = r * t[2 * h:2 * h + 1]
    return r


def _dfl_acc_kernel(logits_ref, label_ref, part_ref, acc_ref, *, eps, ignore_lb):
    t = pl.program_id(1)
    num_t = pl.num_programs(1)

    @pl.when(t == 0)
    def _():
        acc_ref[...] = jnp.zeros_like(acc_ref)

    x = logits_ref[0].astype(jnp.float32)
    lbl = label_ref[0]

    m = jnp.max(x, axis=0, keepdims=True)
    e = jnp.exp(x - m)
    s = jnp.sum(e, axis=0, keepdims=True)
    p = e * (1.0 / s)
    cidx = jax.lax.broadcasted_iota(jnp.int32, x.shape, 0)
    tgt = (cidx == lbl).astype(jnp.float32)
    d = p - tgt
    term = (eps + 1.0) - d * d
    prod = _prod_rows(term)
    loss = -jnp.log(prod)
    valid = lbl != ignore_lb

    acc_ref[0:1, :] += jnp.where(valid, loss, 0.0)
    acc_ref[1:2, :] += valid.astype(jnp.float32)

    @pl.when(t == num_t - 1)
    def _():
        a = acc_ref[...]
        r = a[:, 0:128]
        for k in range(1, a.shape[1] // 128):
            r = r + a[:, k * 128:(k + 1) * 128]
        part_ref[0] = r


def _finalize_kernel(part_ref, out_ref):
    a = part_ref[...]
    ls = jnp.sum(a[:, 0, :], axis=0, keepdims=True)
    nv = jnp.sum(a[:, 1, :], axis=0, keepdims=True)
    ls = jnp.sum(ls, axis=1, keepdims=True)
    nv = jnp.sum(nv, axis=1, keepdims=True)
    out_ref[...] = ls / nv


def _dual_focal_loss_mean(logits, label, *, ignore_lb, eps, tile_p):
    N, C, H, W = logits.shape
    HW = H * W

    max_cols = pl.cdiv(HW, 128) * 128
    tile_p = max(128, min((tile_p // 128) * 128, max_cols))
    HW_pad = pl.cdiv(HW, tile_p) * tile_p
    num_tiles = HW_pad // tile_p

    x = logits.reshape(N, C, HW)
    lbl = label.astype(jnp.int32).reshape(N, 1, HW)
    if HW_pad != HW:
        x = jnp.pad(x, ((0, 0), (0, 0), (0, HW_pad - HW)))
        lbl = jnp.pad(lbl, ((0, 0), (0, 0), (0, HW_pad - HW)),
                      constant_values=ignore_lb)

    partials = pl.pallas_call(
        functools.partial(_dfl_acc_kernel, eps=eps, ignore_lb=ignore_lb),
        out_shape=jax.ShapeDtypeStruct((N, 2, 128), jnp.float32),
        grid_spec=pltpu.PrefetchScalarGridSpec(
            num_scalar_prefetch=0,
            grid=(N, num_tiles),
            in_specs=[
                pl.BlockSpec((1, C, tile_p), lambda n, t: (n, 0, t)),
                pl.BlockSpec((1, 1, tile_p), lambda n, t: (n, 0, t)),
            ],
            out_specs=pl.BlockSpec((1, 2, 128), lambda n, t: (n, 0, 0)),
            scratch_shapes=[pltpu.VMEM((2, tile_p), jnp.float32)],
        ),
        compiler_params=pltpu.CompilerParams(
            dimension_semantics=("parallel", "arbitrary")),
    )(x, lbl)

    out = pl.pallas_call(
        _finalize_kernel,
        out_shape=jax.ShapeDtypeStruct((1, 1), jnp.float32),
    )(partials)
    return out[0, 0]


def kernel(logits, label):
    return _dual_focal_loss_mean(logits, label, ignore_lb=255, eps=1e-5,
                                 tile_p=8192)


# speedup vs baseline: 3.0028x; 2.3713x over previous
import functools

import jax
import jax.numpy as jnp
from jax.experimental import pallas as pl
from jax.experimental.pallas import tpu as pltpu


def _prod_slices(t):
    C = t.shape[0]
    if C == 1:
        return t[0]
    h = C // 2
    r = _prod_slices(t[0:h] * t[h:2 * h])
    if C % 2:
        r = r * t[C - 1]
    return r


def _dfl_acc_kernel(logits_ref, label_ref, part_ref, acc_ref, *, eps, ignore_lb):
    t = pl.program_id(1)
    num_t = pl.num_programs(1)

    @pl.when(t == 0)
    def _():
        acc_ref[...] = jnp.zeros_like(acc_ref)

    x = logits_ref[0].astype(jnp.float32)
    lbl = label_ref[...]

    m = jnp.max(x, axis=0, keepdims=True)
    e = jnp.exp(x - m)
    s = jnp.sum(e, axis=0, keepdims=True)
    p = e * (1.0 / s)
    cidx = jax.lax.broadcasted_iota(jnp.int32, x.shape, 0)
    tgt = (cidx == lbl).astype(jnp.float32)
    d = p - tgt
    term = (eps + 1.0) - d * d
    prod = _prod_slices(term)
    loss = -jnp.log(prod)
    valid = lbl[0] != ignore_lb

    acc_ref[0] += jnp.where(valid, loss, 0.0)
    acc_ref[1] += valid.astype(jnp.float32)

    @pl.when(t == num_t - 1)
    def _():
        a = acc_ref[...]
        Hb, W = a.shape[1], a.shape[2]
        r = a[:, 0:8, :]
        for k in range(1, Hb // 8):
            r = r + a[:, k * 8:(k + 1) * 8, :]
        wf = 128 if W % 128 == 0 else W
        q = r[:, :, 0:wf]
        for k in range(1, W // wf):
            q = q + r[:, :, k * wf:(k + 1) * wf]
        part_ref[0] = q


def _finalize_kernel(part_ref, out_ref):
    a = part_ref[...]
    ls = a[0, 0]
    nv = a[0, 1]
    for n in range(1, a.shape[0]):
        ls = ls + a[n, 0]
        nv = nv + a[n, 1]
    ls = jnp.sum(ls, axis=0, keepdims=True)
    nv = jnp.sum(nv, axis=0, keepdims=True)
    ls = jnp.sum(ls, axis=1, keepdims=True)
    nv = jnp.sum(nv, axis=1, keepdims=True)
    out_ref[...] = ls / nv


def _dual_focal_loss_mean(logits, label, *, ignore_lb, eps, tile_h):
    N, C, H, W = logits.shape

    th = max(8, (tile_h // 8) * 8)
    while H % th:
        th -= 8
        if th < 8:
            th = H
            break
    num_tiles = H // th

    lbl = label.astype(jnp.int32)
    wf = 128 if W % 128 == 0 else W

    partials = pl.pallas_call(
        functools.partial(_dfl_acc_kernel, eps=eps, ignore_lb=ignore_lb),
        out_shape=jax.ShapeDtypeStruct((N, 2, 8, wf), jnp.float32),
        grid_spec=pltpu.PrefetchScalarGridSpec(
            num_scalar_prefetch=0,
            grid=(N, num_tiles),
            in_specs=[
                pl.BlockSpec((1, C, th, W), lambda n, t: (n, 0, t, 0)),
                pl.BlockSpec((1, th, W), lambda n, t: (n, t, 0)),
            ],
            out_specs=pl.BlockSpec((1, 2, 8, wf), lambda n, t: (n, 0, 0, 0)),
            scratch_shapes=[pltpu.VMEM((2, th, W), jnp.float32)],
        ),
        compiler_params=pltpu.CompilerParams(
            dimension_semantics=("parallel", "arbitrary")),
    )(logits, lbl)

    out = pl.pallas_call(
        _finalize_kernel,
        out_shape=jax.ShapeDtypeStruct((1, 1), jnp.float32),
    )(partials)
    return out[0, 0]


def kernel(logits, label):
    return _dual_focal_loss_mean(logits, label, ignore_lb=255, eps=1e-5,
                                 tile_h=32)


# speedup vs baseline: 4.5000x; 1.4986x over previous
import functools

import jax
import jax.numpy as jnp
from jax.experimental import pallas as pl
from jax.experimental.pallas import tpu as pltpu


def _prod_slices(t):
    C = t.shape[0]
    if C == 1:
        return t[0]
    h = C // 2
    r = _prod_slices(t[0:h] * t[h:2 * h])
    if C % 2:
        r = r * t[C - 1]
    return r


def _dfl_block_kernel(logits_ref, label_ref, part_ref, *, eps, ignore_lb, wf):
    x = logits_ref[0].astype(jnp.float32)
    lbl = label_ref[...]

    m = jnp.max(x, axis=0, keepdims=True)
    e = jnp.exp(x - m)
    s = jnp.sum(e, axis=0, keepdims=True)
    p = e * (1.0 / s)
    cidx = jax.lax.broadcasted_iota(jnp.int32, x.shape, 0)
    tgt = (cidx == lbl).astype(jnp.float32)
    d = p - tgt
    term = (eps + 1.0) - d * d
    prod = _prod_slices(term)
    loss = -jnp.log(prod)
    valid = lbl[0] != ignore_lb

    loss = jnp.where(valid, loss, 0.0)
    cnt = valid.astype(jnp.float32)

    th, W = loss.shape
    for arr, row in ((loss, 0), (cnt, 1)):
        r = arr[0:8, :]
        for k in range(1, th // 8):
            r = r + arr[k * 8:(k + 1) * 8, :]
        q = r[:, 0:wf]
        for k in range(1, W // wf):
            q = q + r[:, k * wf:(k + 1) * wf]
        part_ref[0, row] = q


def _finalize_kernel(part_ref, out_ref):
    a = part_ref[...]
    ls = a[0, 0]
    nv = a[0, 1]
    for g in range(1, a.shape[0]):
        ls = ls + a[g, 0]
        nv = nv + a[g, 1]
    ls = jnp.sum(ls, axis=0, keepdims=True)
    nv = jnp.sum(nv, axis=0, keepdims=True)
    ls = jnp.sum(ls, axis=1, keepdims=True)
    nv = jnp.sum(nv, axis=1, keepdims=True)
    out_ref[...] = ls / nv


def _dual_focal_loss_mean(logits, label, *, ignore_lb, eps, tile_h):
    N, C, H, W = logits.shape

    th = max(8, (tile_h // 8) * 8)
    while H % th:
        th -= 8
        if th < 8:
            th = H
            break
    splits = H // th
    G = N * splits

    lbl = label.astype(jnp.int32)
    wf = 128 if W % 128 == 0 else W

    partials = pl.pallas_call(
        functools.partial(_dfl_block_kernel, eps=eps, ignore_lb=ignore_lb,
                          wf=wf),
        out_shape=jax.ShapeDtypeStruct((G, 2, 8, wf), jnp.float32),
        grid_spec=pltpu.PrefetchScalarGridSpec(
            num_scalar_prefetch=0,
            grid=(G,),
            in_specs=[
                pl.BlockSpec((1, C, th, W),
                             lambda i: (i // splits, 0, i % splits, 0)),
                pl.BlockSpec((1, th, W),
                             lambda i: (i // splits, i % splits, 0)),
            ],
            out_specs=pl.BlockSpec((1, 2, 8, wf), lambda i: (i, 0, 0, 0)),
        ),
        compiler_params=pltpu.CompilerParams(
            dimension_semantics=("parallel",)),
    )(logits, lbl)

    out = pl.pallas_call(
        _finalize_kernel,
        out_shape=jax.ShapeDtypeStruct((1, 1), jnp.float32),
    )(partials)
    return out[0, 0]


def kernel(logits, label):
    return _dual_focal_loss_mean(logits, label, ignore_lb=255, eps=1e-5,
                                 tile_h=128)
